# baseline (speedup 1.0000x reference)
import jax
import jax.numpy as jnp
from jax.experimental import pallas as pl
from jax.experimental.pallas import tpu as pltpu

_NCHUNK = 4


def _dma_copy2_kernel(x_ref, a_ref, b_ref, sem_a, sem_b):
    B = x_ref.shape[0]
    c = B // _NCHUNK
    copies = []
    for i in range(_NCHUNK):
        sl = pl.ds(i * c, c)
        ca = pltpu.make_async_copy(x_ref.at[sl], a_ref.at[sl], sem_a.at[i])
        cb = pltpu.make_async_copy(x_ref.at[sl], b_ref.at[sl], sem_b.at[i])
        ca.start()
        cb.start()
        copies.append((ca, cb))
    for ca, cb in copies:
        ca.wait()
        cb.wait()


def kernel(x, mask):
    out = pl.pallas_call(
        _dma_copy2_kernel,
        in_specs=[pl.BlockSpec(memory_space=pl.ANY)],
        out_specs=[
            pl.BlockSpec(memory_space=pl.ANY),
            pl.BlockSpec(memory_space=pl.ANY),
        ],
        out_shape=[
            jax.ShapeDtypeStruct(x.shape, x.dtype),
            jax.ShapeDtypeStruct(x.shape, x.dtype),
        ],
        scratch_shapes=[
            pltpu.SemaphoreType.DMA((_NCHUNK,)),
            pltpu.SemaphoreType.DMA((_NCHUNK,)),
        ],
    )(x)
    return (out[0], out[1])


# speedup vs baseline: 36.9439x; 36.9439x over previous
import functools

import jax
import jax.numpy as jnp
from jax import lax
from jax.experimental import pallas as pl
from jax.experimental.pallas import tpu as pltpu
from jax.experimental.pallas import tpu_sc as plsc

_NC = 2
_NS = 16
_NW = _NC * _NS


def kernel(x, mask):
    B, S, D = x.shape
    R = B * S
    xf = x.reshape(R, D)
    rows_per_w = R // _NW
    n_chunks = 4
    rows_per_chunk = rows_per_w // n_chunks

    mesh = plsc.VectorSubcoreMesh(core_axis_name="c", subcore_axis_name="s")

    @functools.partial(
        pl.kernel,
        mesh=mesh,
        out_type=[
            jax.ShapeDtypeStruct((R, D), x.dtype),
            jax.ShapeDtypeStruct((R, D), x.dtype),
        ],
        scratch_types=[pltpu.VMEM((rows_per_chunk, D), x.dtype)],
    )
    def sc_copy2(x_hbm, a_hbm, b_hbm, buf):
        wid = lax.axis_index("s") * _NC + lax.axis_index("c")
        base = wid * rows_per_w
        for c in range(n_chunks):
            sl = pl.ds(base + c * rows_per_chunk, rows_per_chunk)
            pltpu.sync_copy(x_hbm.at[sl], buf)
            pltpu.sync_copy(buf, a_hbm.at[sl])
            pltpu.sync_copy(buf, b_hbm.at[sl])

    a, b = sc_copy2(xf)
    return (a.reshape(B, S, D), b.reshape(B, S, D))
